# Initial kernel scaffold; baseline (speedup 1.0000x reference)
import jax
import jax.numpy as jnp
from jax.experimental import pallas as pl


def kernel(x, center, W0, g0, b0, W1, g1, b1, W2, g2, b2, W3, g3, b3, Wf, gf, bf):
    raise NotImplementedError("write your pallas kernel here")


# speedup vs baseline: 5.0103x; 5.0103x over previous
import functools

import jax
import jax.numpy as jnp
from jax import lax
from jax.experimental import pallas as pl
from jax.experimental.pallas import tpu as pltpu
from jax.experimental.pallas import tpu_sc as plsc

KNN = 20
BN_ROWS = 512
PB = 128
SB_ROWS = 1024
LW = 128


def _extract_topk(work, idx_ref, b, n):
    iota = lax.broadcasted_iota(jnp.int32, work.shape, 1)
    for t in range(KNN):
        m = jnp.max(work, axis=1, keepdims=True)
        amax = jnp.min(jnp.where(work == m, iota, n), axis=1, keepdims=True)
        idx_ref[:, t:t + 1] = amax + b * n
        work = jnp.where(iota == amax, -jnp.inf, work)


def _topk_body(h_rows_ref, h_all_ref, xxr_ref, xxc_ref, idx_ref):
    b = pl.program_id(0)
    rows = h_rows_ref[...]
    alls = h_all_ref[...]
    g = lax.dot_general(rows, alls, (((1,), (1,)), ((), ())),
                        precision=lax.Precision.DEFAULT,
                        preferred_element_type=jnp.float32)
    work = 2.0 * g - xxr_ref[...] - xxc_ref[0]
    _extract_topk(work, idx_ref, b, work.shape[1])


def _topk(h_pm, batches, n):
    rows_total, d = h_pm.shape
    nb = n // BN_ROWS
    xb = h_pm.reshape(batches, n, d).transpose(0, 2, 1)
    xx = jnp.sum(xb ** 2, axis=1)
    return pl.pallas_call(
        _topk_body,
        grid=(batches, nb),
        in_specs=[
            pl.BlockSpec((BN_ROWS, d), lambda b, i: (b * nb + i, 0)),
            pl.BlockSpec((n, d), lambda b, i: (b, 0)),
            pl.BlockSpec((BN_ROWS, 1), lambda b, i: (b * nb + i, 0)),
            pl.BlockSpec((1, 1, n), lambda b, i: (b, 0, 0)),
        ],
        out_specs=pl.BlockSpec((BN_ROWS, KNN), lambda b, i: (b * nb + i, 0)),
        out_shape=jax.ShapeDtypeStruct((rows_total, KNN), jnp.int32),
    )(h_pm, h_pm, xx.reshape(rows_total, 1), xx.reshape(batches, 1, n))


def _extract_body(pw_ref, idx_ref):
    b = pl.program_id(0)
    work = pw_ref[...]
    _extract_topk(work, idx_ref, b, work.shape[1])


def _extract(pwf, batches, n):
    rows_total = pwf.shape[0]
    nb = n // BN_ROWS
    return pl.pallas_call(
        _extract_body,
        grid=(batches, nb),
        in_specs=[pl.BlockSpec((BN_ROWS, n), lambda b, i: (b * nb + i, 0))],
        out_specs=pl.BlockSpec((BN_ROWS, KNN), lambda b, i: (b * nb + i, 0)),
        out_shape=jax.ShapeDtypeStruct((rows_total, KNN), jnp.int32),
    )(pwf)


def _make_gather_cat(rows_total, dd, cw):
    info = plsc.get_sparse_core_info()
    nw = info.num_cores * info.num_subcores
    pw = rows_total // nw
    c = 4 if cw <= LW else 2
    nchunk = pw // c
    mesh = plsc.VectorSubcoreMesh(core_axis_name="c", subcore_axis_name="s")
    oshape = jax.ShapeDtypeStruct((rows_total * KNN, cw), jnp.float32)

    @functools.partial(
        pl.kernel,
        out_type=oshape,
        mesh=mesh,
        scratch_types=[
            pltpu.VMEM((c * KNN,), jnp.int32),
            pltpu.VMEM((c * KNN,), jnp.int32),
            pltpu.VMEM((c * KNN, LW), jnp.float32),
            pltpu.VMEM((c, LW), jnp.float32),
            pltpu.VMEM((c * KNN, cw), jnp.float32),
            pltpu.SemaphoreType.DMA,
            pltpu.SemaphoreType.DMA,
        ],
    )
    def gather_cat(h_hbm, idx_hbm, pos_hbm, out_hbm,
                   idx_v, pos_v, gath_v, own_v, cat_v, sem_g, sem_s):
        wid = lax.axis_index("s") * info.num_cores + lax.axis_index("c")
        base = wid * pw

        def chunk(gi, carry):
            n0 = base + gi * c
            pltpu.sync_copy(idx_hbm.at[pl.ds(n0 * KNN, c * KNN)], idx_v)
            pltpu.sync_copy(pos_hbm.at[pl.ds(n0 * KNN, c * KNN)], pos_v)
            pltpu.sync_copy(h_hbm.at[pl.ds(n0, c)], own_v)
            pltpu.async_copy(h_hbm.at[idx_v], gath_v, sem_g).wait()

            def point(ci, carry2):
                r0 = ci * KNN
                for fi in range(dd // 16):
                    sl = pl.ds(fi * 16, 16)
                    sr = pl.ds(dd + fi * 16, 16)
                    o = own_v[ci, sl]
                    for k in range(KNN):
                        cat_v[r0 + k, sl] = gath_v[r0 + k, sl] - o
                        cat_v[r0 + k, sr] = o
                return carry2

            lax.fori_loop(0, c, point, 0)
            pltpu.async_copy(cat_v, out_hbm.at[pos_v], sem_s).wait()
            return carry

        lax.fori_loop(0, nchunk, chunk, 0)

    return gather_cat


def _scatter_pos(rows_total):
    i = jnp.arange(rows_total, dtype=jnp.int32)[:, None]
    k = jnp.arange(KNN, dtype=jnp.int32)[None, :]
    return ((i // PB) * (PB * KNN) + k * PB + i % PB).reshape(-1)


def _econv_body(tw, cat_ref, wc_ref, mx_ref, sum_ref):
    i = pl.program_id(0)
    wc = wc_ref[...]
    mx = None
    for k in range(KNN):
        xk = cat_ref[pl.ds(k * PB, PB), :tw]
        yk = lax.dot_general(xk, wc, (((1,), (1,)), ((), ())),
                             precision=lax.Precision.DEFAULT,
                             preferred_element_type=jnp.float32)
        if mx is None:
            mx, s, ss = yk, yk, yk * yk
        else:
            mx = jnp.maximum(mx, yk)
            s = s + yk
            ss = ss + yk * yk
    mx_ref[...] = mx
    t1 = jnp.sum(s, axis=0, keepdims=True)
    t2 = jnp.sum(ss, axis=0, keepdims=True)
    both = jnp.concatenate([t1, t2], axis=0)

    @pl.when(i == 0)
    def _():
        sum_ref[...] = both

    @pl.when(i > 0)
    def _():
        sum_ref[...] += both


def _econv(cat, wcat, rows_total, tw):
    f, cw = wcat.shape[0], cat.shape[1]
    nb = rows_total // PB
    return pl.pallas_call(
        functools.partial(_econv_body, tw),
        grid=(nb,),
        in_specs=[
            pl.BlockSpec((PB * KNN, cw), lambda i: (i, 0)),
            pl.BlockSpec((f, tw), lambda i: (0, 0)),
        ],
        out_specs=[
            pl.BlockSpec((PB, f), lambda i: (i, 0)),
            pl.BlockSpec((2, f), lambda i: (0, 0)),
        ],
        out_shape=[
            jax.ShapeDtypeStruct((rows_total, f), jnp.float32),
            jax.ShapeDtypeStruct((2, f), jnp.float32),
        ],
    )(cat, wcat)


def _norm_body(count, mx_ref, sum_ref, g_ref, b_ref, out_ref):
    m = sum_ref[0:1, :] * (1.0 / count)
    ey2 = sum_ref[1:2, :] * (1.0 / count)
    v = ey2 - m * m
    inv = lax.rsqrt(v + 1e-5)
    y = (mx_ref[...] - m) * (inv * g_ref[...]) + b_ref[...]
    out_ref[...] = jnp.where(y > 0, y, 0.2 * y)


def _norm(mx, sums, gamma, beta, count):
    rows_total, f = mx.shape
    nb = rows_total // SB_ROWS
    return pl.pallas_call(
        functools.partial(_norm_body, count),
        grid=(nb,),
        in_specs=[
            pl.BlockSpec((SB_ROWS, f), lambda i: (i, 0)),
            pl.BlockSpec((2, f), lambda i: (0, 0)),
            pl.BlockSpec((1, f), lambda i: (0, 0)),
            pl.BlockSpec((1, f), lambda i: (0, 0)),
        ],
        out_specs=pl.BlockSpec((SB_ROWS, f), lambda i: (i, 0)),
        out_shape=jax.ShapeDtypeStruct((rows_total, f), jnp.float32),
    )(mx, sums, gamma, beta)


def _fc_body(z_ref, w_ref, y_ref, sum_ref):
    i = pl.program_id(0)
    y = lax.dot_general(z_ref[...], w_ref[...],
                        (((1,), (1,)), ((), ())),
                        precision=lax.Precision.DEFAULT,
                        preferred_element_type=jnp.float32)
    y_ref[...] = y
    t1 = jnp.sum(y, axis=0, keepdims=True)
    t2 = jnp.sum(y * y, axis=0, keepdims=True)
    both = jnp.concatenate([t1, t2], axis=0)

    @pl.when(i == 0)
    def _():
        sum_ref[...] = both

    @pl.when(i > 0)
    def _():
        sum_ref[...] += both


def _final_conv(z, wf):
    rows_total, cin = z.shape
    f = wf.shape[0]
    nb = rows_total // SB_ROWS
    return pl.pallas_call(
        _fc_body,
        grid=(nb,),
        in_specs=[
            pl.BlockSpec((SB_ROWS, cin), lambda i: (i, 0)),
            pl.BlockSpec((f, cin), lambda i: (0, 0)),
        ],
        out_specs=[
            pl.BlockSpec((SB_ROWS, f), lambda i: (i, 0)),
            pl.BlockSpec((2, f), lambda i: (0, 0)),
        ],
        out_shape=[
            jax.ShapeDtypeStruct((rows_total, f), jnp.float32),
            jax.ShapeDtypeStruct((2, f), jnp.float32),
        ],
    )(z, wf)


def _edge_layer(h_pm, w, gamma, beta, batches, n, pos):
    rows_total = h_pm.shape[0]
    f = w.shape[0]
    din = w.shape[1] // 2
    dd = max(din, 16)
    tw = 2 * dd
    cw = max(tw, LW)
    fpad = max(f, LW)
    wcat = jnp.zeros((fpad, tw), jnp.float32)
    wcat = wcat.at[:f, :din].set(w[:, :din])
    wcat = wcat.at[:f, dd:dd + din].set(w[:, din:])
    gp = jnp.pad(gamma, (0, fpad - f), constant_values=1.0).reshape(1, fpad)
    bp = jnp.pad(beta, (0, fpad - f)).reshape(1, fpad)

    idx = _topk(h_pm, batches, n)

    cat = _make_gather_cat(rows_total, dd, cw)(h_pm, idx.reshape(-1), pos)
    mx, sums = _econv(cat, wcat, rows_total, tw)
    count = float(rows_total * KNN)
    return _norm(mx, sums, gp, bp, count)


def kernel(x, center, W0, g0, b0, W1, g1, b1, W2, g2, b2, W3, g3, b3,
           Wf, gf, bf):
    batches, d0, n = x.shape
    rows_total = batches * n
    h = jnp.transpose(x, (0, 2, 1)).reshape(rows_total, d0)
    h = jnp.pad(h, ((0, 0), (0, LW - d0)))
    pos = _scatter_pos(rows_total)
    feats = []
    for w, g, b in ((W0, g0, b0), (W1, g1, b1), (W2, g2, b2), (W3, g3, b3)):
        h = _edge_layer(h, w, g, b, batches, n, pos)
        feats.append(h[:, :w.shape[0]])
    z = jnp.concatenate(feats, axis=1)
    f = Wf.shape[0]
    y, sums = _final_conv(z, Wf)
    out = _norm(y, sums, gf.reshape(1, f), bf.reshape(1, f),
                float(rows_total))
    return jnp.transpose(out.reshape(batches, n, f), (0, 2, 1))


# speedup vs baseline: 7.0988x; 1.4168x over previous
import functools

import jax
import jax.numpy as jnp
from jax import lax
from jax.experimental import pallas as pl
from jax.experimental.pallas import tpu as pltpu
from jax.experimental.pallas import tpu_sc as plsc

KNN = 20
BN_ROWS = 512
PB = 128
SB_ROWS = 1024
LW = 128


def _extract_topk(work, idx_ref, b, n):
    iota = lax.broadcasted_iota(jnp.int32, work.shape, 1)
    for t in range(KNN):
        m = jnp.max(work, axis=1, keepdims=True)
        amax = jnp.min(jnp.where(work == m, iota, n), axis=1, keepdims=True)
        idx_ref[:, t:t + 1] = amax + b * n
        work = jnp.where(iota == amax, -jnp.inf, work)


def _topk_body(h_rows_ref, h_all_ref, xxr_ref, xxc_ref, idx_ref):
    b = pl.program_id(0)
    rows = h_rows_ref[...]
    alls = h_all_ref[...]
    g = lax.dot_general(rows, alls, (((1,), (1,)), ((), ())),
                        precision=lax.Precision.DEFAULT,
                        preferred_element_type=jnp.float32)
    work = 2.0 * g - xxr_ref[...] - xxc_ref[0]
    _extract_topk(work, idx_ref, b, work.shape[1])


def _topk(h_pm, batches, n):
    rows_total, d = h_pm.shape
    nb = n // BN_ROWS
    xb = h_pm.reshape(batches, n, d).transpose(0, 2, 1)
    xx = jnp.sum(xb ** 2, axis=1)
    return pl.pallas_call(
        _topk_body,
        grid=(batches, nb),
        in_specs=[
            pl.BlockSpec((BN_ROWS, d), lambda b, i: (b * nb + i, 0)),
            pl.BlockSpec((n, d), lambda b, i: (b, 0)),
            pl.BlockSpec((BN_ROWS, 1), lambda b, i: (b * nb + i, 0)),
            pl.BlockSpec((1, 1, n), lambda b, i: (b, 0, 0)),
        ],
        out_specs=pl.BlockSpec((BN_ROWS, KNN), lambda b, i: (b * nb + i, 0)),
        out_shape=jax.ShapeDtypeStruct((rows_total, KNN), jnp.int32),
    )(h_pm, h_pm, xx.reshape(rows_total, 1), xx.reshape(batches, 1, n))


def _extract_body(pw_ref, idx_ref):
    b = pl.program_id(0)
    work = pw_ref[...]
    _extract_topk(work, idx_ref, b, work.shape[1])


def _extract(pwf, batches, n):
    rows_total = pwf.shape[0]
    nb = n // BN_ROWS
    return pl.pallas_call(
        _extract_body,
        grid=(batches, nb),
        in_specs=[pl.BlockSpec((BN_ROWS, n), lambda b, i: (b * nb + i, 0))],
        out_specs=pl.BlockSpec((BN_ROWS, KNN), lambda b, i: (b * nb + i, 0)),
        out_shape=jax.ShapeDtypeStruct((rows_total, KNN), jnp.int32),
    )(pwf)


def _make_gather(rows_total):
    info = plsc.get_sparse_core_info()
    nw = info.num_cores * info.num_subcores
    total = rows_total * KNN
    ow = total // nw
    cr = 256
    nchunk = ow // cr
    mesh = plsc.VectorSubcoreMesh(core_axis_name="c", subcore_axis_name="s")
    oshape = jax.ShapeDtypeStruct((total, LW), jnp.float32)

    @functools.partial(
        pl.kernel,
        out_type=oshape,
        mesh=mesh,
        scratch_types=[
            pltpu.VMEM((cr,), jnp.int32),
            pltpu.VMEM((cr, LW), jnp.float32),
            pltpu.SemaphoreType.DMA,
        ],
    )
    def gather(h_hbm, idxp_hbm, out_hbm, idx_v, gath_v, sem_g):
        wid = lax.axis_index("s") * info.num_cores + lax.axis_index("c")
        base = wid * ow

        def chunk(gi, carry):
            r0 = base + gi * cr
            pltpu.sync_copy(idxp_hbm.at[pl.ds(r0, cr)], idx_v)
            pltpu.async_copy(h_hbm.at[idx_v], gath_v, sem_g).wait()
            pltpu.sync_copy(gath_v, out_hbm.at[pl.ds(r0, cr)])
            return carry

        lax.fori_loop(0, nchunk, chunk, 0)

    return gather


def _econv_body(tw, dd, nbr_ref, h_ref, wc_ref, mx_ref, sum_ref):
    i = pl.program_id(0)
    wc = wc_ref[...]
    xi = h_ref[:, :dd]
    mx = None
    for k in range(KNN):
        xj = nbr_ref[pl.ds(k * PB, PB), :dd]
        xk = jnp.concatenate([xj - xi, xi], axis=1)
        yk = lax.dot_general(xk, wc, (((1,), (1,)), ((), ())),
                             precision=lax.Precision.DEFAULT,
                             preferred_element_type=jnp.float32)
        if mx is None:
            mx, s, ss = yk, yk, yk * yk
        else:
            mx = jnp.maximum(mx, yk)
            s = s + yk
            ss = ss + yk * yk
    mx_ref[...] = mx
    t1 = jnp.sum(s, axis=0, keepdims=True)
    t2 = jnp.sum(ss, axis=0, keepdims=True)
    both = jnp.concatenate([t1, t2], axis=0)

    @pl.when(i == 0)
    def _():
        sum_ref[...] = both

    @pl.when(i > 0)
    def _():
        sum_ref[...] += both


def _econv(nbr, h_pm, wcat, rows_total, tw, dd):
    f = wcat.shape[0]
    nb = rows_total // PB
    return pl.pallas_call(
        functools.partial(_econv_body, tw, dd),
        grid=(nb,),
        in_specs=[
            pl.BlockSpec((PB * KNN, LW), lambda i: (i, 0)),
            pl.BlockSpec((PB, LW), lambda i: (i, 0)),
            pl.BlockSpec((f, tw), lambda i: (0, 0)),
        ],
        out_specs=[
            pl.BlockSpec((PB, f), lambda i: (i, 0)),
            pl.BlockSpec((2, f), lambda i: (0, 0)),
        ],
        out_shape=[
            jax.ShapeDtypeStruct((rows_total, f), jnp.float32),
            jax.ShapeDtypeStruct((2, f), jnp.float32),
        ],
    )(nbr, h_pm, wcat)


def _norm_body(count, mx_ref, sum_ref, g_ref, b_ref, out_ref):
    m = sum_ref[0:1, :] * (1.0 / count)
    ey2 = sum_ref[1:2, :] * (1.0 / count)
    v = ey2 - m * m
    inv = lax.rsqrt(v + 1e-5)
    y = (mx_ref[...] - m) * (inv * g_ref[...]) + b_ref[...]
    out_ref[...] = jnp.where(y > 0, y, 0.2 * y)


def _norm(mx, sums, gamma, beta, count):
    rows_total, f = mx.shape
    nb = rows_total // SB_ROWS
    return pl.pallas_call(
        functools.partial(_norm_body, count),
        grid=(nb,),
        in_specs=[
            pl.BlockSpec((SB_ROWS, f), lambda i: (i, 0)),
            pl.BlockSpec((2, f), lambda i: (0, 0)),
            pl.BlockSpec((1, f), lambda i: (0, 0)),
            pl.BlockSpec((1, f), lambda i: (0, 0)),
        ],
        out_specs=pl.BlockSpec((SB_ROWS, f), lambda i: (i, 0)),
        out_shape=jax.ShapeDtypeStruct((rows_total, f), jnp.float32),
    )(mx, sums, gamma, beta)


def _fc_body(z_ref, w_ref, y_ref, sum_ref):
    i = pl.program_id(0)
    y = lax.dot_general(z_ref[...], w_ref[...],
                        (((1,), (1,)), ((), ())),
                        precision=lax.Precision.DEFAULT,
                        preferred_element_type=jnp.float32)
    y_ref[...] = y
    t1 = jnp.sum(y, axis=0, keepdims=True)
    t2 = jnp.sum(y * y, axis=0, keepdims=True)
    both = jnp.concatenate([t1, t2], axis=0)

    @pl.when(i == 0)
    def _():
        sum_ref[...] = both

    @pl.when(i > 0)
    def _():
        sum_ref[...] += both


def _final_conv(z, wf):
    rows_total, cin = z.shape
    f = wf.shape[0]
    nb = rows_total // SB_ROWS
    return pl.pallas_call(
        _fc_body,
        grid=(nb,),
        in_specs=[
            pl.BlockSpec((SB_ROWS, cin), lambda i: (i, 0)),
            pl.BlockSpec((f, cin), lambda i: (0, 0)),
        ],
        out_specs=[
            pl.BlockSpec((SB_ROWS, f), lambda i: (i, 0)),
            pl.BlockSpec((2, f), lambda i: (0, 0)),
        ],
        out_shape=[
            jax.ShapeDtypeStruct((rows_total, f), jnp.float32),
            jax.ShapeDtypeStruct((2, f), jnp.float32),
        ],
    )(z, wf)


def _edge_layer(h_pm, w, gamma, beta, batches, n):
    rows_total = h_pm.shape[0]
    f = w.shape[0]
    din = w.shape[1] // 2
    dd = max(din, 16)
    tw = 2 * dd
    fpad = max(f, LW)
    wcat = jnp.zeros((fpad, tw), jnp.float32)
    wcat = wcat.at[:f, :din].set(w[:, :din])
    wcat = wcat.at[:f, dd:dd + din].set(w[:, din:])
    gp = jnp.pad(gamma, (0, fpad - f), constant_values=1.0).reshape(1, fpad)
    bp = jnp.pad(beta, (0, fpad - f)).reshape(1, fpad)

    idx = _topk(h_pm, batches, n)
    idxp = idx.reshape(rows_total // PB, PB, KNN).transpose(0, 2, 1)
    nbr = _make_gather(rows_total)(h_pm, idxp.reshape(-1))
    mx, sums = _econv(nbr, h_pm, wcat, rows_total, tw, dd)
    count = float(rows_total * KNN)
    return _norm(mx, sums, gp, bp, count)


def kernel(x, center, W0, g0, b0, W1, g1, b1, W2, g2, b2, W3, g3, b3,
           Wf, gf, bf):
    batches, d0, n = x.shape
    rows_total = batches * n
    h = jnp.transpose(x, (0, 2, 1)).reshape(rows_total, d0)
    h = jnp.pad(h, ((0, 0), (0, LW - d0)))
    feats = []
    for w, g, b in ((W0, g0, b0), (W1, g1, b1), (W2, g2, b2), (W3, g3, b3)):
        h = _edge_layer(h, w, g, b, batches, n)
        feats.append(h[:, :w.shape[0]])
    z = jnp.concatenate(feats, axis=1)
    f = Wf.shape[0]
    y, sums = _final_conv(z, Wf)
    out = _norm(y, sums, gf.reshape(1, f), bf.reshape(1, f),
                float(rows_total))
    return jnp.transpose(out.reshape(batches, n, f), (0, 2, 1))


# speedup vs baseline: 7.1316x; 1.0046x over previous
import functools

import jax
import jax.numpy as jnp
from jax import lax
from jax.experimental import pallas as pl
from jax.experimental.pallas import tpu as pltpu
from jax.experimental.pallas import tpu_sc as plsc

KNN = 20
BN_ROWS = 512
PB = 128
SB_ROWS = 1024
LW = 128


GW = 128


def _extract_topk(work, idx_ref, b, n):
    rows = work.shape[0]
    ng = n // GW
    lane = lax.broadcasted_iota(jnp.int32, (rows, GW), 1)
    giota = lax.broadcasted_iota(jnp.int32, (rows, ng), 1)
    tiles = [work[:, g * GW:(g + 1) * GW] for g in range(ng)]
    gm = jnp.concatenate(
        [jnp.max(t, axis=1, keepdims=True) for t in tiles], axis=1)
    removed = []
    for t in range(KNN):
        m = jnp.max(gm, axis=1, keepdims=True)
        wg = jnp.min(jnp.where(gm == m, giota, ng), axis=1, keepdims=True)
        tile = tiles[0]
        for g in range(1, ng):
            tile = jnp.where(wg == g, tiles[g], tile)
        base = wg * GW
        for r in removed:
            tile = jnp.where(lane == r - base, -jnp.inf, tile)
        lidx = jnp.min(jnp.where(tile == m, lane, GW), axis=1, keepdims=True)
        gidx = base + lidx
        idx_ref[:, t:t + 1] = gidx + b * n
        removed.append(gidx)
        gnew = jnp.max(jnp.where(lane == lidx, -jnp.inf, tile),
                       axis=1, keepdims=True)
        gm = jnp.where(giota == wg, gnew, gm)


def _topk_body(h_rows_ref, h_all_ref, xxr_ref, xxc_ref, idx_ref):
    b = pl.program_id(0)
    rows = h_rows_ref[...]
    alls = h_all_ref[...]
    g = lax.dot_general(rows, alls, (((1,), (1,)), ((), ())),
                        precision=lax.Precision.DEFAULT,
                        preferred_element_type=jnp.float32)
    work = 2.0 * g - xxr_ref[...] - xxc_ref[0]
    _extract_topk(work, idx_ref, b, work.shape[1])


def _topk(h_pm, batches, n):
    rows_total, d = h_pm.shape
    nb = n // BN_ROWS
    xb = h_pm.reshape(batches, n, d).transpose(0, 2, 1)
    xx = jnp.sum(xb ** 2, axis=1)
    return pl.pallas_call(
        _topk_body,
        grid=(batches, nb),
        in_specs=[
            pl.BlockSpec((BN_ROWS, d), lambda b, i: (b * nb + i, 0)),
            pl.BlockSpec((n, d), lambda b, i: (b, 0)),
            pl.BlockSpec((BN_ROWS, 1), lambda b, i: (b * nb + i, 0)),
            pl.BlockSpec((1, 1, n), lambda b, i: (b, 0, 0)),
        ],
        out_specs=pl.BlockSpec((BN_ROWS, KNN), lambda b, i: (b * nb + i, 0)),
        out_shape=jax.ShapeDtypeStruct((rows_total, KNN), jnp.int32),
    )(h_pm, h_pm, xx.reshape(rows_total, 1), xx.reshape(batches, 1, n))


def _make_gather(rows_total):
    info = plsc.get_sparse_core_info()
    nw = info.num_cores * info.num_subcores
    total = rows_total * KNN
    ow = total // nw
    cr = 256
    nchunk = ow // cr
    mesh = plsc.VectorSubcoreMesh(core_axis_name="c", subcore_axis_name="s")
    oshape = jax.ShapeDtypeStruct((total, LW), jnp.float32)

    @functools.partial(
        pl.kernel,
        out_type=oshape,
        mesh=mesh,
        scratch_types=[
            pltpu.VMEM((cr,), jnp.int32),
            pltpu.VMEM((cr, LW), jnp.float32),
            pltpu.SemaphoreType.DMA,
        ],
    )
    def gather(h_hbm, idxp_hbm, out_hbm, idx_v, gath_v, sem_g):
        wid = lax.axis_index("s") * info.num_cores + lax.axis_index("c")
        base = wid * ow

        def chunk(gi, carry):
            r0 = base + gi * cr
            pltpu.sync_copy(idxp_hbm.at[pl.ds(r0, cr)], idx_v)
            pltpu.async_copy(h_hbm.at[idx_v], gath_v, sem_g).wait()
            pltpu.sync_copy(gath_v, out_hbm.at[pl.ds(r0, cr)])
            return carry

        lax.fori_loop(0, nchunk, chunk, 0)

    return gather


def _econv_body(tw, dd, nbr_ref, h_ref, wc_ref, mx_ref, sum_ref):
    i = pl.program_id(0)
    wc = wc_ref[...]
    xi = h_ref[:, :dd]
    mx = None
    for k in range(KNN):
        xj = nbr_ref[pl.ds(k * PB, PB), :dd]
        xk = jnp.concatenate([xj - xi, xi], axis=1)
        yk = lax.dot_general(xk, wc, (((1,), (1,)), ((), ())),
                             precision=lax.Precision.DEFAULT,
                             preferred_element_type=jnp.float32)
        if mx is None:
            mx, s, ss = yk, yk, yk * yk
        else:
            mx = jnp.maximum(mx, yk)
            s = s + yk
            ss = ss + yk * yk
    mx_ref[...] = mx
    t1 = jnp.sum(s, axis=0, keepdims=True)
    t2 = jnp.sum(ss, axis=0, keepdims=True)
    both = jnp.concatenate([t1, t2], axis=0)

    @pl.when(i == 0)
    def _():
        sum_ref[...] = both

    @pl.when(i > 0)
    def _():
        sum_ref[...] += both


def _econv(nbr, h_pm, wcat, rows_total, tw, dd):
    f = wcat.shape[0]
    nb = rows_total // PB
    return pl.pallas_call(
        functools.partial(_econv_body, tw, dd),
        grid=(nb,),
        in_specs=[
            pl.BlockSpec((PB * KNN, LW), lambda i: (i, 0)),
            pl.BlockSpec((PB, LW), lambda i: (i, 0)),
            pl.BlockSpec((f, tw), lambda i: (0, 0)),
        ],
        out_specs=[
            pl.BlockSpec((PB, f), lambda i: (i, 0)),
            pl.BlockSpec((2, f), lambda i: (0, 0)),
        ],
        out_shape=[
            jax.ShapeDtypeStruct((rows_total, f), jnp.float32),
            jax.ShapeDtypeStruct((2, f), jnp.float32),
        ],
    )(nbr, h_pm, wcat)


def _norm_body(count, mx_ref, sum_ref, g_ref, b_ref, out_ref):
    m = sum_ref[0:1, :] * (1.0 / count)
    ey2 = sum_ref[1:2, :] * (1.0 / count)
    v = ey2 - m * m
    inv = lax.rsqrt(v + 1e-5)
    y = (mx_ref[...] - m) * (inv * g_ref[...]) + b_ref[...]
    out_ref[...] = jnp.where(y > 0, y, 0.2 * y)


def _norm(mx, sums, gamma, beta, count):
    rows_total, f = mx.shape
    nb = rows_total // SB_ROWS
    return pl.pallas_call(
        functools.partial(_norm_body, count),
        grid=(nb,),
        in_specs=[
            pl.BlockSpec((SB_ROWS, f), lambda i: (i, 0)),
            pl.BlockSpec((2, f), lambda i: (0, 0)),
            pl.BlockSpec((1, f), lambda i: (0, 0)),
            pl.BlockSpec((1, f), lambda i: (0, 0)),
        ],
        out_specs=pl.BlockSpec((SB_ROWS, f), lambda i: (i, 0)),
        out_shape=jax.ShapeDtypeStruct((rows_total, f), jnp.float32),
    )(mx, sums, gamma, beta)


def _fc_body(z_ref, w_ref, y_ref, sum_ref):
    i = pl.program_id(0)
    y = lax.dot_general(z_ref[...], w_ref[...],
                        (((1,), (1,)), ((), ())),
                        precision=lax.Precision.DEFAULT,
                        preferred_element_type=jnp.float32)
    y_ref[...] = y
    t1 = jnp.sum(y, axis=0, keepdims=True)
    t2 = jnp.sum(y * y, axis=0, keepdims=True)
    both = jnp.concatenate([t1, t2], axis=0)

    @pl.when(i == 0)
    def _():
        sum_ref[...] = both

    @pl.when(i > 0)
    def _():
        sum_ref[...] += both


def _final_conv(z, wf):
    rows_total, cin = z.shape
    f = wf.shape[0]
    nb = rows_total // SB_ROWS
    return pl.pallas_call(
        _fc_body,
        grid=(nb,),
        in_specs=[
            pl.BlockSpec((SB_ROWS, cin), lambda i: (i, 0)),
            pl.BlockSpec((f, cin), lambda i: (0, 0)),
        ],
        out_specs=[
            pl.BlockSpec((SB_ROWS, f), lambda i: (i, 0)),
            pl.BlockSpec((2, f), lambda i: (0, 0)),
        ],
        out_shape=[
            jax.ShapeDtypeStruct((rows_total, f), jnp.float32),
            jax.ShapeDtypeStruct((2, f), jnp.float32),
        ],
    )(z, wf)


def _edge_layer(h_pm, w, gamma, beta, batches, n):
    rows_total = h_pm.shape[0]
    f = w.shape[0]
    din = w.shape[1] // 2
    dd = max(din, 16)
    tw = 2 * dd
    fpad = max(f, LW)
    wcat = jnp.zeros((fpad, tw), jnp.float32)
    wcat = wcat.at[:f, :din].set(w[:, :din])
    wcat = wcat.at[:f, dd:dd + din].set(w[:, din:])
    gp = jnp.pad(gamma, (0, fpad - f), constant_values=1.0).reshape(1, fpad)
    bp = jnp.pad(beta, (0, fpad - f)).reshape(1, fpad)

    idx = _topk(h_pm, batches, n)
    idxp = idx.reshape(rows_total // PB, PB, KNN).transpose(0, 2, 1)
    nbr = _make_gather(rows_total)(h_pm, idxp.reshape(-1))
    mx, sums = _econv(nbr, h_pm, wcat, rows_total, tw, dd)
    count = float(rows_total * KNN)
    return _norm(mx, sums, gp, bp, count)


def kernel(x, center, W0, g0, b0, W1, g1, b1, W2, g2, b2, W3, g3, b3,
           Wf, gf, bf):
    batches, d0, n = x.shape
    rows_total = batches * n
    h = jnp.transpose(x, (0, 2, 1)).reshape(rows_total, d0)
    h = jnp.pad(h, ((0, 0), (0, LW - d0)))
    feats = []
    for w, g, b in ((W0, g0, b0), (W1, g1, b1), (W2, g2, b2), (W3, g3, b3)):
        h = _edge_layer(h, w, g, b, batches, n)
        feats.append(h[:, :w.shape[0]])
    z = jnp.concatenate(feats, axis=1)
    f = Wf.shape[0]
    y, sums = _final_conv(z, Wf)
    out = _norm(y, sums, gf.reshape(1, f), bf.reshape(1, f),
                float(rows_total))
    return jnp.transpose(out.reshape(batches, n, f), (0, 2, 1))
